# baseline (speedup 1.0000x reference)
import functools

import jax
import jax.numpy as jnp
from jax.experimental import pallas as pl
from jax.experimental.pallas import tpu as pltpu

N_TRAIN = 100000
D = 16
N_CLASSES = 100
K = 8
Q = 1024

NB = 4096
NSUB = 8
SB = NB // NSUB
NBLK = (N_TRAIN + NB - 1) // NB
N_PAD = NBLK * NB
CW = NSUB * K

_BIGCODE = float(N_PAD * 128 + 256)
_INF = jnp.inf
_PADV = 1e10


def _knn_kernel(x_ref, txt_ref, ty_ref, out_ref, d_ref, cv_ref, cc_ref,
                rv_ref, rc_ref):
    j = pl.program_id(0)

    x = x_ref[...]
    t = txt_ref[...]
    xt = jnp.dot(x, t, preferred_element_type=jnp.float32)
    x_sq = jnp.sum(x * x, axis=1, keepdims=True)
    t_sq = jnp.sum(t * t, axis=0, keepdims=True)
    d = x_sq - 2.0 * xt + t_sq

    lidx = jax.lax.broadcasted_iota(jnp.int32, (1, NB), 1)
    gidx = j * NB + lidx
    lab = ty_ref[...].reshape(1, NB)
    code = (gidx * 128 + lab).astype(jnp.float32)

    @pl.when(j == 0)
    def _init():
        rv_ref[...] = jnp.full((Q, K), _INF, jnp.float32)
        rc_ref[...] = jnp.full((Q, K), _BIGCODE, jnp.float32) + \
            jax.lax.broadcasted_iota(jnp.int32, (Q, K), 1).astype(jnp.float32)

    rv7 = rv_ref[:, K - 1:K]
    iota8 = jax.lax.broadcasted_iota(jnp.int32, (Q, K), 1)

    tmaxs = []
    for s in range(NSUB):
        ds = d[:, s * SB:(s + 1) * SB]
        cs = code[:, s * SB:(s + 1) * SB]
        cnt = jnp.sum((ds < rv7).astype(jnp.int32), axis=1, keepdims=True)
        tmaxs.append(jnp.minimum(jnp.max(cnt), K))
        m1 = jnp.min(ds, axis=1, keepdims=True)
        c1 = jnp.min(jnp.where(ds <= m1, cs, _BIGCODE), axis=1,
                     keepdims=True)
        cv_ref[:, s * K:(s + 1) * K] = jnp.where(iota8 == 0, m1, _INF)
        cc_ref[:, s * K:(s + 1) * K] = jnp.where(iota8 == 0, c1, _BIGCODE)
        d_ref[:, s * SB:(s + 1) * SB] = jnp.where(cs == c1, _INF, ds)

    for i in range(1, K):
        for s in range(NSUB):
            @pl.when(i < tmaxs[s])
            def _extract(i=i, s=s):
                dd = d_ref[:, s * SB:(s + 1) * SB]
                cs = code[:, s * SB:(s + 1) * SB]
                m = jnp.min(dd, axis=1, keepdims=True)
                c = jnp.min(jnp.where(dd <= m, cs, _BIGCODE), axis=1,
                            keepdims=True)
                cv_ref[:, s * K + i:s * K + i + 1] = m
                cc_ref[:, s * K + i:s * K + i + 1] = c
                if i + 1 < K:
                    @pl.when(i + 1 < tmaxs[s])
                    def _mask():
                        d_ref[:, s * SB:(s + 1) * SB] = \
                            jnp.where(cs == c, _INF, dd)

    vw = jnp.concatenate([rv_ref[...], cv_ref[...]], axis=1)
    cw = jnp.concatenate([rc_ref[...], cc_ref[...]], axis=1)
    nv, nc = [], []
    for _ in range(K):
        m = jnp.min(vw, axis=1, keepdims=True)
        c = jnp.min(jnp.where(vw <= m, cw, _BIGCODE), axis=1,
                    keepdims=True)
        nv.append(m)
        nc.append(c)
        vw = jnp.where(cw == c, _INF, vw)
    rv_ref[...] = jnp.concatenate(nv, axis=1)
    rc_ref[...] = jnp.concatenate(nc, axis=1)

    @pl.when(j == NBLK - 1)
    def _finalize():
        codes = rc_ref[...]
        labf = codes - 128.0 * jnp.floor(codes * (1.0 / 128.0))
        labs = labf.astype(jnp.int32)
        ci = jax.lax.broadcasted_iota(jnp.int32, (Q, 128), 1)
        votes = jnp.zeros((Q, 128), jnp.float32)
        for k in range(K):
            votes = votes + (labs[:, k:k + 1] == ci).astype(jnp.float32)
        mv = jnp.max(votes, axis=1, keepdims=True)
        pred = jnp.min(jnp.where(votes >= mv, ci, 1 << 20), axis=1,
                       keepdims=True)
        out_ref[...] = pred


@functools.partial(jax.jit, static_argnames=())
def kernel(x, train_x, train_y):
    txt = jnp.pad(train_x.T, ((0, 0), (0, N_PAD - N_TRAIN)),
                  constant_values=_PADV)
    ty = jnp.pad(train_y.astype(jnp.int32), (0, N_PAD - N_TRAIN))
    ty3 = ty.reshape(NBLK, 1, NB)

    out = pl.pallas_call(
        _knn_kernel,
        grid=(NBLK,),
        in_specs=[
            pl.BlockSpec((Q, D), lambda j: (0, 0)),
            pl.BlockSpec((D, NB), lambda j: (0, j)),
            pl.BlockSpec((1, 1, NB), lambda j: (j, 0, 0)),
        ],
        out_specs=pl.BlockSpec((Q, 1), lambda j: (0, 0)),
        out_shape=jax.ShapeDtypeStruct((Q, 1), jnp.int32),
        scratch_shapes=[
            pltpu.VMEM((Q, NB), jnp.float32),
            pltpu.VMEM((Q, CW), jnp.float32),
            pltpu.VMEM((Q, CW), jnp.float32),
            pltpu.VMEM((Q, K), jnp.float32),
            pltpu.VMEM((Q, K), jnp.float32),
        ],
        compiler_params=pltpu.CompilerParams(
            dimension_semantics=("arbitrary",),
        ),
    )(x, txt, ty3)
    return out[:, 0]


# speedup vs baseline: 1.9021x; 1.9021x over previous
import functools

import jax
import jax.numpy as jnp
from jax.experimental import pallas as pl
from jax.experimental.pallas import tpu as pltpu

N_TRAIN = 100000
D = 16
N_CLASSES = 100
K = 8
Q = 1024

NB = 2048
NSUB = 4
SB = NB // NSUB
NBLK = (N_TRAIN + NB - 1) // NB
N_PAD = NBLK * NB
CW = NSUB * K

_BIGCODE = float(N_PAD * 128 + 256)
_INF = jnp.inf
_PADV = 1e10


def _knn_kernel(x_ref, txt_ref, ty_ref, out_ref, d_ref, cv_ref, cc_ref,
                rv_ref, rc_ref):
    j = pl.program_id(0)

    x = x_ref[...]
    t = txt_ref[...]
    xt = jnp.dot(x, t, preferred_element_type=jnp.float32)
    x_sq = jnp.sum(x * x, axis=1, keepdims=True)
    t_sq = jnp.sum(t * t, axis=0, keepdims=True)
    d = x_sq - 2.0 * xt + t_sq

    lidx = jax.lax.broadcasted_iota(jnp.int32, (1, NB), 1)
    gidx = j * NB + lidx
    lab = ty_ref[...].reshape(1, NB)
    code = (gidx * 128 + lab).astype(jnp.float32)

    @pl.when(j == 0)
    def _init():
        rv_ref[...] = jnp.full((Q, K), _INF, jnp.float32)
        rc_ref[...] = jnp.full((Q, K), _BIGCODE, jnp.float32) + \
            jax.lax.broadcasted_iota(jnp.int32, (Q, K), 1).astype(jnp.float32)

    rv7 = rv_ref[:, K - 1:K]
    iota8 = jax.lax.broadcasted_iota(jnp.int32, (Q, K), 1)

    tmaxs = []
    for s in range(NSUB):
        ds = d[:, s * SB:(s + 1) * SB]
        cs = code[:, s * SB:(s + 1) * SB]
        cnt = jnp.sum((ds < rv7).astype(jnp.int32), axis=1, keepdims=True)
        tmaxs.append(jnp.minimum(jnp.max(cnt), K))
        m1 = jnp.min(ds, axis=1, keepdims=True)
        c1 = jnp.min(jnp.where(ds <= m1, cs, _BIGCODE), axis=1,
                     keepdims=True)
        cv_ref[:, s * K:(s + 1) * K] = jnp.where(iota8 == 0, m1, _INF)
        cc_ref[:, s * K:(s + 1) * K] = jnp.where(iota8 == 0, c1, _BIGCODE)
        d_ref[:, s * SB:(s + 1) * SB] = jnp.where(cs == c1, _INF, ds)

    for i in range(1, K):
        for s in range(NSUB):
            @pl.when(i < tmaxs[s])
            def _extract(i=i, s=s):
                dd = d_ref[:, s * SB:(s + 1) * SB]
                cs = code[:, s * SB:(s + 1) * SB]
                m = jnp.min(dd, axis=1, keepdims=True)
                c = jnp.min(jnp.where(dd <= m, cs, _BIGCODE), axis=1,
                            keepdims=True)
                cv_ref[:, s * K + i:s * K + i + 1] = m
                cc_ref[:, s * K + i:s * K + i + 1] = c
                if i + 1 < K:
                    @pl.when(i + 1 < tmaxs[s])
                    def _mask():
                        d_ref[:, s * SB:(s + 1) * SB] = \
                            jnp.where(cs == c, _INF, dd)

    vw = jnp.concatenate([rv_ref[...], cv_ref[...]], axis=1)
    cw = jnp.concatenate([rc_ref[...], cc_ref[...]], axis=1)
    nv, nc = [], []
    for _ in range(K):
        m = jnp.min(vw, axis=1, keepdims=True)
        c = jnp.min(jnp.where(vw <= m, cw, _BIGCODE), axis=1,
                    keepdims=True)
        nv.append(m)
        nc.append(c)
        vw = jnp.where(cw == c, _INF, vw)
    rv_ref[...] = jnp.concatenate(nv, axis=1)
    rc_ref[...] = jnp.concatenate(nc, axis=1)

    @pl.when(j == NBLK - 1)
    def _finalize():
        codes = rc_ref[...]
        labf = codes - 128.0 * jnp.floor(codes * (1.0 / 128.0))
        labs = labf.astype(jnp.int32)
        ci = jax.lax.broadcasted_iota(jnp.int32, (Q, 128), 1)
        votes = jnp.zeros((Q, 128), jnp.float32)
        for k in range(K):
            votes = votes + (labs[:, k:k + 1] == ci).astype(jnp.float32)
        mv = jnp.max(votes, axis=1, keepdims=True)
        pred = jnp.min(jnp.where(votes >= mv, ci, 1 << 20), axis=1,
                       keepdims=True)
        out_ref[...] = pred


@functools.partial(jax.jit, static_argnames=())
def kernel(x, train_x, train_y):
    txt = jnp.pad(train_x.T, ((0, 0), (0, N_PAD - N_TRAIN)),
                  constant_values=_PADV)
    ty = jnp.pad(train_y.astype(jnp.int32), (0, N_PAD - N_TRAIN))
    ty3 = ty.reshape(NBLK, 1, NB)

    out = pl.pallas_call(
        _knn_kernel,
        grid=(NBLK,),
        in_specs=[
            pl.BlockSpec((Q, D), lambda j: (0, 0)),
            pl.BlockSpec((D, NB), lambda j: (0, j)),
            pl.BlockSpec((1, 1, NB), lambda j: (j, 0, 0)),
        ],
        out_specs=pl.BlockSpec((Q, 1), lambda j: (0, 0)),
        out_shape=jax.ShapeDtypeStruct((Q, 1), jnp.int32),
        scratch_shapes=[
            pltpu.VMEM((Q, NB), jnp.float32),
            pltpu.VMEM((Q, CW), jnp.float32),
            pltpu.VMEM((Q, CW), jnp.float32),
            pltpu.VMEM((Q, K), jnp.float32),
            pltpu.VMEM((Q, K), jnp.float32),
        ],
        compiler_params=pltpu.CompilerParams(
            dimension_semantics=("arbitrary",),
        ),
    )(x, txt, ty3)
    return out[:, 0]


# speedup vs baseline: 1.9045x; 1.0013x over previous
import functools

import jax
import jax.numpy as jnp
from jax.experimental import pallas as pl
from jax.experimental.pallas import tpu as pltpu

N_TRAIN = 100000
D = 16
N_CLASSES = 100
K = 8
Q = 1024

NB = 2048
NSUB = 4
SB = NB // NSUB
NBLK = (N_TRAIN + NB - 1) // NB
N_PAD = NBLK * NB
CW = NSUB * K

_BIGCODE = float(N_PAD * 128 + 256)
_INF = jnp.inf
_PADV = 1e10


def _knn_kernel(x_ref, txt_ref, ty_ref, out_ref, d_ref, cv_ref, cc_ref,
                rv_ref, rc_ref):
    j = pl.program_id(0)

    x = x_ref[...]
    t = txt_ref[...]
    xt2 = jnp.dot(x * -2.0, t, preferred_element_type=jnp.float32)
    x_sq = jnp.sum(x * x, axis=1, keepdims=True)
    t_sq = jnp.sum(t * t, axis=0, keepdims=True)
    d = x_sq + xt2 + t_sq

    lidx = jax.lax.broadcasted_iota(jnp.int32, (1, NB), 1)
    gidx = j * NB + lidx
    lab = ty_ref[...].reshape(1, NB)
    code = (gidx * 128 + lab).astype(jnp.float32)

    @pl.when(j == 0)
    def _init():
        rv_ref[...] = jnp.full((Q, K), _INF, jnp.float32)
        rc_ref[...] = jnp.full((Q, K), _BIGCODE, jnp.float32) + \
            jax.lax.broadcasted_iota(jnp.int32, (Q, K), 1).astype(jnp.float32)

    rv7 = rv_ref[:, K - 1:K]
    iota8 = jax.lax.broadcasted_iota(jnp.int32, (Q, K), 1)

    tmaxs = []
    for s in range(NSUB):
        ds = d[:, s * SB:(s + 1) * SB]
        cs = code[:, s * SB:(s + 1) * SB]
        cnt = jnp.sum((ds < rv7).astype(jnp.int32), axis=1, keepdims=True)
        tmaxs.append(jnp.minimum(jnp.max(cnt), K))
        m1 = jnp.min(ds, axis=1, keepdims=True)
        c1 = jnp.min(jnp.where(ds <= m1, cs, _BIGCODE), axis=1,
                     keepdims=True)
        cv_ref[:, s * K:(s + 1) * K] = jnp.where(iota8 == 0, m1, _INF)
        cc_ref[:, s * K:(s + 1) * K] = jnp.where(iota8 == 0, c1, _BIGCODE)
        d_ref[:, s * SB:(s + 1) * SB] = jnp.where(cs == c1, _INF, ds)

    for i in range(1, K):
        for s in range(NSUB):
            @pl.when(i < tmaxs[s])
            def _extract(i=i, s=s):
                dd = d_ref[:, s * SB:(s + 1) * SB]
                cs = code[:, s * SB:(s + 1) * SB]
                m = jnp.min(dd, axis=1, keepdims=True)
                c = jnp.min(jnp.where(dd <= m, cs, _BIGCODE), axis=1,
                            keepdims=True)
                cv_ref[:, s * K + i:s * K + i + 1] = m
                cc_ref[:, s * K + i:s * K + i + 1] = c
                if i + 1 < K:
                    @pl.when(i + 1 < tmaxs[s])
                    def _mask():
                        d_ref[:, s * SB:(s + 1) * SB] = \
                            jnp.where(cs == c, _INF, dd)

    vw = jnp.concatenate([rv_ref[...], cv_ref[...]], axis=1)
    cw = jnp.concatenate([rc_ref[...], cc_ref[...]], axis=1)
    nv, nc = [], []
    for _ in range(K):
        m = jnp.min(vw, axis=1, keepdims=True)
        c = jnp.min(jnp.where(vw <= m, cw, _BIGCODE), axis=1,
                    keepdims=True)
        nv.append(m)
        nc.append(c)
        vw = jnp.where(cw == c, _INF, vw)
    rv_ref[...] = jnp.concatenate(nv, axis=1)
    rc_ref[...] = jnp.concatenate(nc, axis=1)

    @pl.when(j == NBLK - 1)
    def _finalize():
        codes = rc_ref[...]
        labf = codes - 128.0 * jnp.floor(codes * (1.0 / 128.0))
        labs = labf.astype(jnp.int32)
        ci = jax.lax.broadcasted_iota(jnp.int32, (Q, 128), 1)
        votes = jnp.zeros((Q, 128), jnp.float32)
        for k in range(K):
            votes = votes + (labs[:, k:k + 1] == ci).astype(jnp.float32)
        mv = jnp.max(votes, axis=1, keepdims=True)
        pred = jnp.min(jnp.where(votes >= mv, ci, 1 << 20), axis=1,
                       keepdims=True)
        out_ref[...] = pred


@functools.partial(jax.jit, static_argnames=())
def kernel(x, train_x, train_y):
    txt = jnp.pad(train_x.T, ((0, 0), (0, N_PAD - N_TRAIN)),
                  constant_values=_PADV)
    ty = jnp.pad(train_y.astype(jnp.int32), (0, N_PAD - N_TRAIN))
    ty3 = ty.reshape(NBLK, 1, NB)

    out = pl.pallas_call(
        _knn_kernel,
        grid=(NBLK,),
        in_specs=[
            pl.BlockSpec((Q, D), lambda j: (0, 0)),
            pl.BlockSpec((D, NB), lambda j: (0, j)),
            pl.BlockSpec((1, 1, NB), lambda j: (j, 0, 0)),
        ],
        out_specs=pl.BlockSpec((Q, 1), lambda j: (0, 0)),
        out_shape=jax.ShapeDtypeStruct((Q, 1), jnp.int32),
        scratch_shapes=[
            pltpu.VMEM((Q, NB), jnp.float32),
            pltpu.VMEM((Q, CW), jnp.float32),
            pltpu.VMEM((Q, CW), jnp.float32),
            pltpu.VMEM((Q, K), jnp.float32),
            pltpu.VMEM((Q, K), jnp.float32),
        ],
        compiler_params=pltpu.CompilerParams(
            dimension_semantics=("arbitrary",),
        ),
    )(x, txt, ty3)
    return out[:, 0]
